# Initial kernel scaffold; baseline (speedup 1.0000x reference)
import jax
import jax.numpy as jnp
from jax.experimental import pallas as pl


def kernel(activations):
    raise NotImplementedError("write your pallas kernel here")


# speedup vs baseline: 4.7940x; 4.7940x over previous
import numpy as np
from math import factorial

import jax
import jax.numpy as jnp
from jax.experimental import pallas as pl

_LMAX = 5
_NTAU = 16


def _cg_coef(l1, l2, l, m1, m2):
    m = m1 + m2
    if abs(m) > l:
        return 0.0
    pref = (2 * l + 1) * factorial(l + l1 - l2) * factorial(l - l1 + l2) * factorial(l1 + l2 - l) / factorial(l1 + l2 + l + 1)
    pref *= factorial(l + m) * factorial(l - m) * factorial(l1 - m1) * factorial(l1 + m1) * factorial(l2 - m2) * factorial(l2 + m2)
    kmin = max(0, l2 - l - m1, l1 + m2 - l)
    kmax = min(l1 + l2 - l, l1 - m1, l2 + m2)
    s = 0.0
    for k in range(kmin, kmax + 1):
        s += (-1) ** k / (factorial(k) * factorial(l1 + l2 - l - k) * factorial(l1 - m1 - k) * factorial(l2 + m2 - k) * factorial(l - l2 + m1 + k) * factorial(l - l1 - m2 + k))
    return float(np.sqrt(pref) * s)


def _ltuples(lmax):
    out = []
    for l in range(lmax + 1):
        pairs = []
        for l1 in range(lmax + 1):
            for l2 in range(l1, lmax + 1):
                if l2 - l1 <= l <= l1 + l2:
                    pairs.append((l1, l2))
        out.append(sorted(pairs))
    return out


def _build_tables():
    lt = _ltuples(_LMAX)
    cum16 = np.concatenate([[0], (1 + 2 * np.arange(_LMAX + 1)).cumsum()]).astype(int)
    nrows = sum((2 * l + 1) * len(lt[l]) for l in range(_LMAX + 1))
    S = np.zeros((nrows, 36 * 36), dtype=np.float32)
    k = 0
    for l in range(_LMAX + 1):
        mats = {}
        for (l1, l2) in lt[l]:
            M = np.zeros((2 * l + 1, 2 * l1 + 1, 2 * l2 + 1), dtype=np.float64)
            for m1 in range(-l1, l1 + 1):
                for m2 in range(-l2, l2 + 1):
                    m = m1 + m2
                    if abs(m) <= l:
                        M[m + l, m1 + l1, m2 + l2] = _cg_coef(l1, l2, l, m1, m2)
            mats[(l1, l2)] = M
        for a in range(2 * l + 1):
            for (l1, l2) in lt[l]:
                M = mats[(l1, l2)]
                for x in range(2 * l1 + 1):
                    for y in range(2 * l2 + 1):
                        c = M[a, x, y]
                        if c != 0.0:
                            S[k, (cum16[l1] + x) * 36 + (cum16[l2] + y)] = c
                k += 1
    assert k == nrows
    REP = np.zeros((16, 512), dtype=np.float32)
    A = np.zeros((16, 512), dtype=np.float32)
    Bm = np.zeros((16, 512), dtype=np.float32)
    for s in range(16):
        REP[s, 32 * s:32 * s + 32] = 1.0
        for t in range(16):
            A[t, 32 * s + 2 * t] = 1.0
            Bm[t, 32 * s + 2 * t + 1] = 1.0
    K1 = np.zeros((1296, 36), dtype=np.float32)
    K2 = np.zeros((1296, 36), dtype=np.float32)
    for u in range(36):
        for v in range(36):
            K1[u * 36 + v, u] = 1.0
            K2[u * 36 + v, v] = 1.0
    return S, REP, A, Bm, K1, K2


_S, _REP, _A, _B, _K1, _K2 = _build_tables()
_NROWS = _S.shape[0]


def _body(act_ref, s_ref, rep_ref, a_ref, b_ref, k1_ref, k2_ref, out_ref):
    x = act_ref[0, 0]
    y = act_ref[0, 1]
    dot = lambda p, q: jax.lax.dot(p, q, preferred_element_type=jnp.float32)
    xe = dot(k1_ref[...], x)
    ye = dot(k1_ref[...], y)
    xt = dot(k2_ref[...], x)
    yt = dot(k2_ref[...], y)
    p1 = dot(xe, rep_ref[...])
    p2 = dot(ye, rep_ref[...])
    qa = dot(xt, a_ref[...]) + dot(yt, b_ref[...])
    qb = dot(xt, b_ref[...]) - dot(yt, a_ref[...])
    zc = p1 * qa + p2 * qb
    out_ref[0] = dot(s_ref[...], zc)


def kernel(activations):
    B = activations.shape[0]
    act = activations.transpose(0, 2, 1).reshape(B, 2, 36, 16)
    out = pl.pallas_call(
        _body,
        grid=(B,),
        in_specs=[
            pl.BlockSpec((1, 2, 36, 16), lambda b: (b, 0, 0, 0)),
            pl.BlockSpec((_NROWS, 1296), lambda b: (0, 0)),
            pl.BlockSpec((16, 512), lambda b: (0, 0)),
            pl.BlockSpec((16, 512), lambda b: (0, 0)),
            pl.BlockSpec((16, 512), lambda b: (0, 0)),
            pl.BlockSpec((1296, 36), lambda b: (0, 0)),
            pl.BlockSpec((1296, 36), lambda b: (0, 0)),
        ],
        out_specs=pl.BlockSpec((1, _NROWS, 512), lambda b: (b, 0, 0)),
        out_shape=jax.ShapeDtypeStruct((B, _NROWS, 512), jnp.float32),
    )(act, _S, _REP, _A, _B, _K1, _K2)
    return out.reshape(B, _NROWS * 256, 2)
